# Initial kernel scaffold; baseline (speedup 1.0000x reference)
import jax
import jax.numpy as jnp
from jax.experimental import pallas as pl


def kernel(x, pos_table):
    raise NotImplementedError("write your pallas kernel here")


# speedup vs baseline: 3.1692x; 3.1692x over previous
import jax
import jax.numpy as jnp
from jax.experimental import pallas as pl

_RB = 1024


def _add_body(x_ref, p_ref, o_ref):
    o_ref[...] = x_ref[...] + p_ref[...]


def kernel(x, pos_table):
    b, s, d = x.shape
    xf = x.reshape(b * s, d)
    ns = s // _RB
    out = pl.pallas_call(
        _add_body,
        grid=(ns, b),
        in_specs=[
            pl.BlockSpec((_RB, d), lambda i, j, ns=ns: (j * ns + i, 0)),
            pl.BlockSpec((_RB, d), lambda i, j: (i, 0)),
        ],
        out_specs=pl.BlockSpec((_RB, d), lambda i, j, ns=ns: (j * ns + i, 0)),
        out_shape=jax.ShapeDtypeStruct((b * s, d), x.dtype),
    )(xf, pos_table)
    return out.reshape(b, s, d)
